# baseline (speedup 1.0000x reference)
import jax
import jax.numpy as jnp
from jax import lax
from jax.experimental import pallas as pl

_NB = 128
_S = 4
_R = 4096


def _pwl_tables(W1, b1, W2, b2):
    a = W1[0]
    c = b1
    W2 = W2.astype(jnp.bfloat16).astype(jnp.float32)
    H = a.shape[0]
    iszero = a == 0
    bp = jnp.where(iszero, jnp.inf, -c / jnp.where(iszero, 1.0, a))
    sgn = jnp.where(iszero, 0.0, jnp.where(a > 0, 1.0, -1.0))
    dA = (sgn * a)[:, None] * W2
    dB = (sgn * c)[:, None] * W2
    neg = (a < 0).astype(jnp.float32)
    base_alpha = ((a * neg)[:, None] * W2).sum(0)
    base_beta = (
        b2
        + ((c * neg)[:, None] * W2).sum(0)
        + ((jnp.maximum(c, 0.0) * iszero)[:, None] * W2).sum(0)
    )
    order = jnp.argsort(bp)
    zero4 = jnp.zeros((1, _S), jnp.float32)
    alpha_t = base_alpha[None] + jnp.concatenate([zero4, jnp.cumsum(dA[order], 0)], 0)
    beta_t = base_beta[None] + jnp.concatenate([zero4, jnp.cumsum(dB[order], 0)], 0)
    bp_pad = jnp.full((1, _NB), jnp.inf, jnp.float32).at[0, :H].set(bp[order])
    coef = jnp.zeros((2 * _S, _NB), jnp.float32)
    coef = coef.at[:_S, : H + 1].set(alpha_t.T)
    coef = coef.at[_S:, : H + 1].set(beta_t.T)
    return bp_pad, coef


def _pwl_block(x_ref, bp_ref, coef_ref, o_ref):
    x = x_ref[...]
    bp = jnp.broadcast_to(bp_ref[...], (_R, _NB))

    b = [bp_ref[0, 16 * i + 15] for i in range(7)]
    c1 = b[3] < x
    bv2 = jnp.where(c1, b[5], b[1])
    c2 = bv2 < x
    bv3 = jnp.where(c1, jnp.where(c2, b[6], b[4]), jnp.where(c2, b[2], b[0]))
    c3 = bv3 < x
    b8 = [bp_ref[0, 16 * i + 7] for i in range(8)]
    bv4 = jnp.where(
        c1,
        jnp.where(c2, jnp.where(c3, b8[7], b8[6]), jnp.where(c3, b8[5], b8[4])),
        jnp.where(c2, jnp.where(c3, b8[3], b8[2]), jnp.where(c3, b8[1], b8[0])),
    )
    c4 = bv4 < x
    b16 = [bp_ref[0, 8 * i + 3] for i in range(16)]
    t = [jnp.where(c4, b16[2 * i + 1], b16[2 * i]) for i in range(8)]
    t = [jnp.where(c3, t[2 * i + 1], t[2 * i]) for i in range(4)]
    t = [jnp.where(c2, t[2 * i + 1], t[2 * i]) for i in range(2)]
    bv5 = jnp.where(c1, t[1], t[0])
    c5 = bv5 < x
    pos = (
        jnp.where(c1, 64, 0)
        + jnp.where(c2, 32, 0)
        + jnp.where(c3, 16, 0)
        + jnp.where(c4, 8, 0)
        + jnp.where(c5, 4, 0)
    )
    for step in (2, 1):
        bv = jnp.take_along_axis(bp, pos + (step - 1), axis=1)
        pos = jnp.where(bv < x, pos + step, pos)

    planes = []
    for k in range(_S):
        ak = jnp.take_along_axis(
            jnp.broadcast_to(coef_ref[k : k + 1, :], (_R, _NB)), pos, axis=1
        )
        bk = jnp.take_along_axis(
            jnp.broadcast_to(coef_ref[k + _S : k + _S + 1, :], (_R, _NB)), pos, axis=1
        )
        planes.append(ak * x + bk)

    o_ref[...] = jnp.stack(planes, axis=1).reshape(_R * _S, 128)

def kernel(rx, phase, W1, b1, W2, b2):
    del phase
    N = rx.shape[0]
    M = N // 128
    bp_pad, coef = _pwl_tables(W1, b1, W2, b2)
    xm = rx.reshape(M, 128)
    out = pl.pallas_call(
        _pwl_block,
        grid=(M // _R,),
        in_specs=[
            pl.BlockSpec((_R, 128), lambda i: (i, 0)),
            pl.BlockSpec((1, _NB), lambda i: (0, 0)),
            pl.BlockSpec((2 * _S, _NB), lambda i: (0, 0)),
        ],
        out_specs=pl.BlockSpec((_R * _S, 128), lambda i: (i, 0)),
        out_shape=jax.ShapeDtypeStruct((M * _S, 128), jnp.float32),
    )(xm, bp_pad, coef)
    return out.reshape(M, _S, 128).transpose(0, 2, 1).reshape(N, _S)


# speedup vs baseline: 1.0000x; 1.0000x over previous
import jax
import jax.numpy as jnp
from jax import lax
from jax.experimental import pallas as pl
from jax.experimental.pallas import tpu as pltpu

_NB = 128
_S = 4
_R = 2048


def _pwl_tables(W1, b1, W2, b2):
    a = W1[0]
    c = b1
    W2 = W2.astype(jnp.bfloat16).astype(jnp.float32)
    H = a.shape[0]
    iszero = a == 0
    bp = jnp.where(iszero, jnp.inf, -c / jnp.where(iszero, 1.0, a))
    sgn = jnp.where(iszero, 0.0, jnp.where(a > 0, 1.0, -1.0))
    dA = (sgn * a)[:, None] * W2
    dB = (sgn * c)[:, None] * W2
    neg = (a < 0).astype(jnp.float32)
    base_alpha = ((a * neg)[:, None] * W2).sum(0)
    base_beta = (
        b2
        + ((c * neg)[:, None] * W2).sum(0)
        + ((jnp.maximum(c, 0.0) * iszero)[:, None] * W2).sum(0)
    )
    order = jnp.argsort(bp)
    zero4 = jnp.zeros((1, _S), jnp.float32)
    alpha_t = base_alpha[None] + jnp.concatenate([zero4, jnp.cumsum(dA[order], 0)], 0)
    beta_t = base_beta[None] + jnp.concatenate([zero4, jnp.cumsum(dB[order], 0)], 0)
    bp_pad = jnp.full((1, _NB), jnp.inf, jnp.float32).at[0, :H].set(bp[order])
    coef = jnp.zeros((2 * _S, _NB), jnp.float32)
    coef = coef.at[:_S, : H + 1].set(alpha_t.T)
    coef = coef.at[_S:, : H + 1].set(beta_t.T)
    return bp_pad, coef


def _pwl_block(x_ref, bp_ref, coef_ref, o_ref):
    x = x_ref[...]
    bp = jnp.broadcast_to(bp_ref[...], (_R, _NB))

    b = [bp_ref[0, 16 * i + 15] for i in range(7)]
    c1 = b[3] < x
    bv2 = jnp.where(c1, b[5], b[1])
    c2 = bv2 < x
    bv3 = jnp.where(c1, jnp.where(c2, b[6], b[4]), jnp.where(c2, b[2], b[0]))
    c3 = bv3 < x
    b8 = [bp_ref[0, 16 * i + 7] for i in range(8)]
    bv4 = jnp.where(
        c1,
        jnp.where(c2, jnp.where(c3, b8[7], b8[6]), jnp.where(c3, b8[5], b8[4])),
        jnp.where(c2, jnp.where(c3, b8[3], b8[2]), jnp.where(c3, b8[1], b8[0])),
    )
    c4 = bv4 < x
    b16 = [bp_ref[0, 8 * i + 3] for i in range(16)]
    t = [jnp.where(c4, b16[2 * i + 1], b16[2 * i]) for i in range(8)]
    t = [jnp.where(c3, t[2 * i + 1], t[2 * i]) for i in range(4)]
    t = [jnp.where(c2, t[2 * i + 1], t[2 * i]) for i in range(2)]
    bv5 = jnp.where(c1, t[1], t[0])
    c5 = bv5 < x
    pos = (
        jnp.where(c1, 64, 0)
        + jnp.where(c2, 32, 0)
        + jnp.where(c3, 16, 0)
        + jnp.where(c4, 8, 0)
        + jnp.where(c5, 4, 0)
    )
    for step in (2, 1):
        bv = jnp.take_along_axis(bp, pos + (step - 1), axis=1)
        pos = jnp.where(bv < x, pos + step, pos)

    planes = []
    for k in range(_S):
        ak = jnp.take_along_axis(
            jnp.broadcast_to(coef_ref[k : k + 1, :], (_R, _NB)), pos, axis=1
        )
        bk = jnp.take_along_axis(
            jnp.broadcast_to(coef_ref[k + _S : k + _S + 1, :], (_R, _NB)), pos, axis=1
        )
        planes.append(ak * x + bk)

    o_ref[...] = jnp.stack(planes, axis=1).reshape(_R * _S, 128)

def kernel(rx, phase, W1, b1, W2, b2):
    del phase
    N = rx.shape[0]
    M = N // 128
    bp_pad, coef = _pwl_tables(W1, b1, W2, b2)
    xm = rx.reshape(M, 128)
    out = pl.pallas_call(
        _pwl_block,
        grid=(M // _R,),
        compiler_params=pltpu.CompilerParams(
            dimension_semantics=("parallel",)
        ),
        in_specs=[
            pl.BlockSpec((_R, 128), lambda i: (i, 0)),
            pl.BlockSpec((1, _NB), lambda i: (0, 0)),
            pl.BlockSpec((2 * _S, _NB), lambda i: (0, 0)),
        ],
        out_specs=pl.BlockSpec((_R * _S, 128), lambda i: (i, 0)),
        out_shape=jax.ShapeDtypeStruct((M * _S, 128), jnp.float32),
    )(xm, bp_pad, coef)
    return out.reshape(M, _S, 128).transpose(0, 2, 1).reshape(N, _S)
